# baseline (speedup 1.0000x reference)
import functools

import jax
import jax.numpy as jnp
from jax import lax
from jax.experimental import pallas as pl
from jax.experimental.pallas import tpu as pltpu
from jax.experimental.pallas import tpu_sc as plsc

_B, _S, _D, _H, _E, _K = 2, 2048, 1024, 1024, 8, 2
_M = _B * _S
_NP = _M * _K
_LBW = 0.01

_TM = 256
_GT = 40
_GMAX = _GT * _TM

_BM_ROUTE = 512

_INTERPRET = False


def _routing_body(x_ref, wg_ref, bg_ref, i1_ref, i2_ref, p1_ref, p2_ref,
                  load_ref, lbl_ref):
    m = pl.program_id(0)
    logits = jnp.dot(x_ref[...], wg_ref[...],
                     preferred_element_type=jnp.float32) + bg_ref[...]
    iota = lax.broadcasted_iota(jnp.int32, logits.shape, 1)
    m1 = jnp.max(logits, axis=-1, keepdims=True)
    i1 = jnp.min(jnp.where(logits == m1, iota, _E), axis=-1, keepdims=True)
    masked = jnp.where(iota == i1, -jnp.inf, logits)
    m2 = jnp.max(masked, axis=-1, keepdims=True)
    i2 = jnp.min(jnp.where(masked == m2, iota, _E), axis=-1, keepdims=True)
    e2 = jnp.exp(m2 - m1)
    p1 = 1.0 / (1.0 + e2)
    p2 = e2 / (1.0 + e2)
    i1_ref[...] = i1
    i2_ref[...] = i2
    p1_ref[...] = p1
    p2_ref[...] = p2
    mask = p1 * (iota == i1).astype(jnp.float32) \
        + p2 * (iota == i2).astype(jnp.float32)
    part = jnp.sum(mask, axis=0, keepdims=True) / float(_M)

    @pl.when(m == 0)
    def _():
        load_ref[...] = part

    @pl.when(m != 0)
    def _():
        load_ref[...] = load_ref[...] + part

    @pl.when(m == pl.num_programs(0) - 1)
    def _():
        lbl_ref[...] = _LBW * jnp.sum((load_ref[...] - 1.0 / _E) ** 2,
                                      keepdims=True)


def _routing(x_flat, wg, bg):
    nblk = _M // _BM_ROUTE
    out_shape = (
        jax.ShapeDtypeStruct((_M, 1), jnp.int32),
        jax.ShapeDtypeStruct((_M, 1), jnp.int32),
        jax.ShapeDtypeStruct((_M, 1), jnp.float32),
        jax.ShapeDtypeStruct((_M, 1), jnp.float32),
        jax.ShapeDtypeStruct((1, _E), jnp.float32),
        jax.ShapeDtypeStruct((1, 1), jnp.float32),
    )
    tok_spec = pl.BlockSpec((_BM_ROUTE, 1), lambda m: (m, 0))
    return pl.pallas_call(
        _routing_body,
        grid=(nblk,),
        in_specs=[
            pl.BlockSpec((_BM_ROUTE, _D), lambda m: (m, 0)),
            pl.BlockSpec((_D, _E), lambda m: (0, 0)),
            pl.BlockSpec((1, _E), lambda m: (0, 0)),
        ],
        out_specs=(tok_spec, tok_spec, tok_spec, tok_spec,
                   pl.BlockSpec((1, _E), lambda m: (0, 0)),
                   pl.BlockSpec((1, 1), lambda m: (0, 0))),
        out_shape=out_shape,
        interpret=_INTERPRET,
    )(x_flat, wg, bg.reshape(1, _E))


def _gmm_body(te_ref, xs_ref, w1_ref, b1_ref, w2_ref, b2_ref, ys_ref,
              w1s_ref, w2s_ref):
    m = pl.program_id(0)
    active = te_ref[m] < _E
    changed = (m == 0) | (te_ref[m] != te_ref[jnp.maximum(m - 1, 0)])

    @pl.when(active & changed)
    def _():
        w1s_ref[...] = w1_ref[0].astype(jnp.bfloat16)
        w2s_ref[...] = w2_ref[0].astype(jnp.bfloat16)

    @pl.when(active)
    def _():
        x = xs_ref[...].astype(jnp.bfloat16)
        h = jnp.dot(x, w1s_ref[...], preferred_element_type=jnp.float32) \
            + b1_ref[0]
        h = 0.5 * h * (1.0 + lax.erf(h * 0.7071067811865476))
        y = jnp.dot(h.astype(jnp.bfloat16), w2s_ref[...],
                    preferred_element_type=jnp.float32) + b2_ref[0]
        ys_ref[...] = y


def _gmm(xs, w1, b1, w2, b2, te):
    def eclamp(m, te):
        return jnp.minimum(te[m], _E - 1)

    grid_spec = pltpu.PrefetchScalarGridSpec(
        num_scalar_prefetch=1,
        grid=(_GT,),
        in_specs=[
            pl.BlockSpec((_TM, _D), lambda m, te: (m, 0)),
            pl.BlockSpec((1, _D, _H), lambda m, te: (eclamp(m, te), 0, 0)),
            pl.BlockSpec((1, 1, _H), lambda m, te: (eclamp(m, te), 0, 0)),
            pl.BlockSpec((1, _H, _D), lambda m, te: (eclamp(m, te), 0, 0)),
            pl.BlockSpec((1, 1, _D), lambda m, te: (eclamp(m, te), 0, 0)),
        ],
        out_specs=pl.BlockSpec((_TM, _D), lambda m, te: (m, 0)),
        scratch_shapes=[
            pltpu.VMEM((_D, _H), jnp.bfloat16),
            pltpu.VMEM((_H, _D), jnp.bfloat16),
        ],
    )
    return pl.pallas_call(
        _gmm_body,
        grid_spec=grid_spec,
        out_shape=jax.ShapeDtypeStruct((_GMAX, _D), jnp.float32),
        interpret=_INTERPRET,
    )(te, xs, w1, b1.reshape(_E, 1, _H), w2, b2.reshape(_E, 1, _D))


_NB = 8
_BT = _M // _NB


def _dispatch_body(i1_ref, i2_ref, pos1_ref, pos2_ref, te_ref):
    i1 = i1_ref[...]
    i2 = i2_ref[...]
    lane = lax.broadcasted_iota(jnp.int32, (_M, _E), 1)
    oh1 = (lane == i1).astype(jnp.float32)
    oh2 = (lane == i2).astype(jnp.float32)
    oh12 = oh1 + oh2
    r_i = lax.broadcasted_iota(jnp.int32, (_BT, _BT), 0)
    c_i = lax.broadcasted_iota(jnp.int32, (_BT, _BT), 1)
    ltri = (c_i < r_i).astype(jnp.float32)
    blocks = []
    colsums = []
    for b in range(_NB):
        ohb = oh12[b * _BT:(b + 1) * _BT, :]
        blocks.append(jnp.dot(ltri, ohb, preferred_element_type=jnp.float32))
        colsums.append(jnp.sum(ohb, axis=0, keepdims=True))
    prefix = jnp.concatenate(blocks, axis=0)
    colsum = jnp.concatenate(colsums, axis=0)
    rb_i = lax.broadcasted_iota(jnp.int32, (_NB, _NB), 0)
    cb_i = lax.broadcasted_iota(jnp.int32, (_NB, _NB), 1)
    ltri_b = (cb_i < rb_i).astype(jnp.float32)
    block_excl = jnp.dot(ltri_b, colsum,
                         preferred_element_type=jnp.float32)
    totals = jnp.sum(colsum, axis=0, keepdims=True)
    acount = jnp.ceil(totals / _TM) * _TM
    re_i = lax.broadcasted_iota(jnp.int32, (_E, _E), 0)
    ce_i = lax.broadcasted_iota(jnp.int32, (_E, _E), 1)
    ltri_e = (re_i < ce_i).astype(jnp.float32)
    acum = jnp.dot(acount, ltri_e,
                   preferred_element_type=jnp.float32)
    blk_off = jnp.concatenate(
        [jnp.broadcast_to(block_excl[b:b + 1, :], (_BT, _E))
         for b in range(_NB)], axis=0)
    base = prefix + blk_off + acum
    pos1_ref[...] = jnp.sum(oh1 * base, axis=1,
                            keepdims=True).astype(jnp.int32)
    pos2_ref[...] = jnp.sum(oh2 * base, axis=1,
                            keepdims=True).astype(jnp.int32)
    tile_start = lax.broadcasted_iota(jnp.int32, (_GT, _E), 0) \
        .astype(jnp.float32) * _TM
    aend = acum + acount
    te_ref[...] = jnp.sum((tile_start >= aend).astype(jnp.int32),
                          axis=1, keepdims=True)


def _dispatch(i1, i2):
    return pl.pallas_call(
        _dispatch_body,
        out_shape=(
            jax.ShapeDtypeStruct((_M, 1), jnp.int32),
            jax.ShapeDtypeStruct((_M, 1), jnp.int32),
            jax.ShapeDtypeStruct((_GT, 1), jnp.int32),
        ),
        interpret=_INTERPRET,
    )(i1, i2)


_NW = 32
_TW = _M // _NW
_CC = 32
_NCH = _TW // _CC


def _sc_mesh():
    return plsc.VectorSubcoreMesh(core_axis_name="c", subcore_axis_name="s")


def _scatter_body(x_hbm, pos1_hbm, pos2_hbm, xs_hbm,
                  posv1, posv2, xbuf, sem):
    wid = lax.axis_index("s") * 2 + lax.axis_index("c")
    base = wid * _TW
    for ch in range(_NCH):
        b = base + ch * _CC
        pltpu.sync_copy(pos1_hbm.at[pl.ds(b, _CC)], posv1)
        pltpu.sync_copy(pos2_hbm.at[pl.ds(b, _CC)], posv2)
        pltpu.sync_copy(x_hbm.at[pl.ds(b, _CC)], xbuf)
        c1 = pltpu.async_copy(xbuf, xs_hbm.at[posv1], sem)
        c2 = pltpu.async_copy(xbuf, xs_hbm.at[posv2], sem)
        c1.wait()
        c2.wait()


def _sc_scatter(x_flat, pos1, pos2):
    return pl.kernel(
        _scatter_body,
        out_type=jax.ShapeDtypeStruct((_GMAX, _D), jnp.float32),
        mesh=_sc_mesh(),
        scratch_types=[
            pltpu.VMEM((_CC,), jnp.int32),
            pltpu.VMEM((_CC,), jnp.int32),
            pltpu.VMEM((_CC, _D), jnp.float32),
            pltpu.SemaphoreType.DMA,
        ],
        interpret=_INTERPRET,
    )(x_flat, pos1, pos2)


def _combine_body(ys_hbm, pos1_hbm, pos2_hbm, p1_hbm, p2_hbm, out_hbm,
                  posv1, posv2, pv1, pv2, buf1, buf2, sem):
    wid = lax.axis_index("s") * 2 + lax.axis_index("c")
    base = wid * _TW
    for ch in range(_NCH):
        b = base + ch * _CC
        pltpu.sync_copy(pos1_hbm.at[pl.ds(b, _CC)], posv1)
        pltpu.sync_copy(pos2_hbm.at[pl.ds(b, _CC)], posv2)
        pltpu.sync_copy(p1_hbm.at[pl.ds(b, _CC)], pv1.at[pl.ds(0, _CC)])
        pltpu.sync_copy(p2_hbm.at[pl.ds(b, _CC)], pv2.at[pl.ds(0, _CC)])
        g1 = pltpu.async_copy(ys_hbm.at[posv1], buf1, sem)
        g2 = pltpu.async_copy(ys_hbm.at[posv2], buf2, sem)
        g1.wait()
        g2.wait()

        def _row(r, _):
            s1 = pv1[pl.ds(r, 16)][0]
            s2 = pv2[pl.ds(r, 16)][0]
            for c in range(_D // 16):
                sl = pl.ds(c * 16, 16)
                buf1[r, sl] = buf1[r, sl] * s1 + buf2[r, sl] * s2
            return 0

        lax.fori_loop(0, _CC, _row, 0)
        pltpu.sync_copy(buf1, out_hbm.at[pl.ds(b, _CC)])


def _sc_combine(ys, pos1, pos2, p1, p2):
    return pl.kernel(
        _combine_body,
        out_type=jax.ShapeDtypeStruct((_M, _D), jnp.float32),
        mesh=_sc_mesh(),
        scratch_types=[
            pltpu.VMEM((_CC,), jnp.int32),
            pltpu.VMEM((_CC,), jnp.int32),
            pltpu.VMEM((_CC + 16,), jnp.float32),
            pltpu.VMEM((_CC + 16,), jnp.float32),
            pltpu.VMEM((_CC, _D), jnp.float32),
            pltpu.VMEM((_CC, _D), jnp.float32),
            pltpu.SemaphoreType.DMA,
        ],
        interpret=_INTERPRET,
    )(ys, pos1, pos2, p1, p2)


def kernel(x, Wg, bg, W1, b1, W2, b2):
    x_flat = x.reshape(_M, _D)
    i1, i2, p1, p2, load, lbl = _routing(x_flat, Wg, bg)
    pos1, pos2, te = _dispatch(i1, i2)
    pos1, pos2 = pos1.reshape(_M), pos2.reshape(_M)
    xs = _sc_scatter(x_flat, pos1, pos2)
    ys = _gmm(xs, W1, b1, W2, b2,
              te.reshape(_GT))
    combined = _sc_combine(ys, pos1, pos2, p1.reshape(_M), p2.reshape(_M))

    return (combined.reshape(_B, _S, _D), lbl.reshape(()), load.reshape(_E))


# speedup vs baseline: 1.0322x; 1.0322x over previous
import jax
import jax.numpy as jnp
from jax import lax
from jax.experimental import pallas as pl
from jax.experimental.pallas import tpu as pltpu
from jax.experimental.pallas import tpu_sc as plsc

_B, _S, _D, _H, _E, _K = 2, 2048, 1024, 1024, 8, 2
_M = _B * _S
_LBW = 0.01

_TM = 256
_GT = 40
_GMAX = _GT * _TM

_BM_ROUTE = 512

_INTERPRET = False


def _routing_body(x_ref, wg_ref, bg_ref, i1_ref, i2_ref, p1_ref, p2_ref,
                  load_ref, lbl_ref):
    m = pl.program_id(0)
    logits = jnp.dot(x_ref[...], wg_ref[...],
                     preferred_element_type=jnp.float32) + bg_ref[...]
    iota = lax.broadcasted_iota(jnp.int32, logits.shape, 1)
    m1 = jnp.max(logits, axis=-1, keepdims=True)
    i1 = jnp.min(jnp.where(logits == m1, iota, _E), axis=-1, keepdims=True)
    masked = jnp.where(iota == i1, -jnp.inf, logits)
    m2 = jnp.max(masked, axis=-1, keepdims=True)
    i2 = jnp.min(jnp.where(masked == m2, iota, _E), axis=-1, keepdims=True)
    e2 = jnp.exp(m2 - m1)
    p1 = 1.0 / (1.0 + e2)
    p2 = e2 / (1.0 + e2)
    i1_ref[...] = i1
    i2_ref[...] = i2
    p1_ref[...] = p1
    p2_ref[...] = p2
    mask = p1 * (iota == i1).astype(jnp.float32) \
        + p2 * (iota == i2).astype(jnp.float32)
    part = jnp.sum(mask, axis=0, keepdims=True) / float(_M)

    @pl.when(m == 0)
    def _():
        load_ref[...] = part

    @pl.when(m != 0)
    def _():
        load_ref[...] = load_ref[...] + part

    @pl.when(m == pl.num_programs(0) - 1)
    def _():
        lbl_ref[...] = _LBW * jnp.sum((load_ref[...] - 1.0 / _E) ** 2,
                                      keepdims=True)


def _routing(x_flat, wg, bg):
    nblk = _M // _BM_ROUTE
    out_shape = (
        jax.ShapeDtypeStruct((_M, 1), jnp.int32),
        jax.ShapeDtypeStruct((_M, 1), jnp.int32),
        jax.ShapeDtypeStruct((_M, 1), jnp.float32),
        jax.ShapeDtypeStruct((_M, 1), jnp.float32),
        jax.ShapeDtypeStruct((1, _E), jnp.float32),
        jax.ShapeDtypeStruct((1, 1), jnp.float32),
    )
    tok_spec = pl.BlockSpec((_BM_ROUTE, 1), lambda m: (m, 0))
    return pl.pallas_call(
        _routing_body,
        grid=(nblk,),
        in_specs=[
            pl.BlockSpec((_BM_ROUTE, _D), lambda m: (m, 0)),
            pl.BlockSpec((_D, _E), lambda m: (0, 0)),
            pl.BlockSpec((1, _E), lambda m: (0, 0)),
        ],
        out_specs=(tok_spec, tok_spec, tok_spec, tok_spec,
                   pl.BlockSpec((1, _E), lambda m: (0, 0)),
                   pl.BlockSpec((1, 1), lambda m: (0, 0))),
        out_shape=out_shape,
        interpret=_INTERPRET,
    )(x_flat, wg, bg.reshape(1, _E))


def _gmm_body(te_ref, xs_ref, w1_ref, b1_ref, w2_ref, b2_ref, ys_ref):
    m = pl.program_id(0)

    @pl.when(te_ref[m] < _E)
    def _():
        x = xs_ref[...]
        h = jnp.dot(x, w1_ref[0], preferred_element_type=jnp.float32) \
            + b1_ref[0]
        h = 0.5 * h * (1.0 + lax.erf(h * 0.7071067811865476))
        y = jnp.dot(h, w2_ref[0],
                    preferred_element_type=jnp.float32) + b2_ref[0]
        ys_ref[...] = y


def _gmm(xs, w1, b1, w2, b2, te):
    def eclamp(m, te):
        return jnp.minimum(te[m], _E - 1)

    grid_spec = pltpu.PrefetchScalarGridSpec(
        num_scalar_prefetch=1,
        grid=(_GT,),
        in_specs=[
            pl.BlockSpec((_TM, _D), lambda m, te: (m, 0)),
            pl.BlockSpec((1, _D, _H), lambda m, te: (eclamp(m, te), 0, 0)),
            pl.BlockSpec((1, 1, _H), lambda m, te: (eclamp(m, te), 0, 0)),
            pl.BlockSpec((1, _H, _D), lambda m, te: (eclamp(m, te), 0, 0)),
            pl.BlockSpec((1, 1, _D), lambda m, te: (eclamp(m, te), 0, 0)),
        ],
        out_specs=pl.BlockSpec((_TM, _D), lambda m, te: (m, 0)),
    )
    return pl.pallas_call(
        _gmm_body,
        grid_spec=grid_spec,
        out_shape=jax.ShapeDtypeStruct((_GMAX, _D), jnp.float32),
        interpret=_INTERPRET,
    )(te, xs, w1, b1.reshape(_E, 1, _H), w2, b2.reshape(_E, 1, _D))


_NB = 8
_BT = _M // _NB


def _dispatch_body(i1_ref, i2_ref, pos1_ref, pos2_ref, te_ref):
    i1 = i1_ref[...]
    i2 = i2_ref[...]
    lane = lax.broadcasted_iota(jnp.int32, (_M, _E), 1)
    oh1 = (lane == i1).astype(jnp.float32)
    oh2 = (lane == i2).astype(jnp.float32)
    oh12 = oh1 + oh2
    r_i = lax.broadcasted_iota(jnp.int32, (_BT, _BT), 0)
    c_i = lax.broadcasted_iota(jnp.int32, (_BT, _BT), 1)
    ltri = (c_i < r_i).astype(jnp.float32)
    blocks = []
    colsums = []
    for b in range(_NB):
        ohb = oh12[b * _BT:(b + 1) * _BT, :]
        blocks.append(jnp.dot(ltri, ohb, preferred_element_type=jnp.float32))
        colsums.append(jnp.sum(ohb, axis=0, keepdims=True))
    prefix = jnp.concatenate(blocks, axis=0)
    colsum = jnp.concatenate(colsums, axis=0)
    rb_i = lax.broadcasted_iota(jnp.int32, (_NB, _NB), 0)
    cb_i = lax.broadcasted_iota(jnp.int32, (_NB, _NB), 1)
    ltri_b = (cb_i < rb_i).astype(jnp.float32)
    block_excl = jnp.dot(ltri_b, colsum,
                         preferred_element_type=jnp.float32)
    totals = jnp.sum(colsum, axis=0, keepdims=True)
    acount = jnp.ceil(totals / _TM) * _TM
    re_i = lax.broadcasted_iota(jnp.int32, (_E, _E), 0)
    ce_i = lax.broadcasted_iota(jnp.int32, (_E, _E), 1)
    ltri_e = (re_i < ce_i).astype(jnp.float32)
    acum = jnp.dot(acount, ltri_e,
                   preferred_element_type=jnp.float32)
    blk_off = jnp.concatenate(
        [jnp.broadcast_to(block_excl[b:b + 1, :], (_BT, _E))
         for b in range(_NB)], axis=0)
    base = prefix + blk_off + acum
    pos1_ref[...] = jnp.sum(oh1 * base, axis=1,
                            keepdims=True).astype(jnp.int32)
    pos2_ref[...] = jnp.sum(oh2 * base, axis=1,
                            keepdims=True).astype(jnp.int32)
    tile_start = lax.broadcasted_iota(jnp.int32, (_GT, _E), 0) \
        .astype(jnp.float32) * _TM
    aend = acum + acount
    te_ref[...] = jnp.sum((tile_start >= aend).astype(jnp.int32),
                          axis=1, keepdims=True)


def _dispatch(i1, i2):
    return pl.pallas_call(
        _dispatch_body,
        out_shape=(
            jax.ShapeDtypeStruct((_M, 1), jnp.int32),
            jax.ShapeDtypeStruct((_M, 1), jnp.int32),
            jax.ShapeDtypeStruct((_GT, 1), jnp.int32),
        ),
        interpret=_INTERPRET,
    )(i1, i2)


_NW = 32
_TW = _M // _NW
_CC = 32
_NCH = _TW // _CC


def _sc_mesh():
    return plsc.VectorSubcoreMesh(core_axis_name="c", subcore_axis_name="s")


def _scatter_body(x_hbm, pos1_hbm, pos2_hbm, xs_hbm,
                  posv1, posv2, xbuf, sem):
    wid = lax.axis_index("s") * 2 + lax.axis_index("c")
    base = wid * _TW
    for ch in range(_NCH):
        b = base + ch * _CC
        pltpu.sync_copy(pos1_hbm.at[pl.ds(b, _CC)], posv1)
        pltpu.sync_copy(pos2_hbm.at[pl.ds(b, _CC)], posv2)
        pltpu.sync_copy(x_hbm.at[pl.ds(b, _CC)], xbuf)
        c1 = pltpu.async_copy(xbuf, xs_hbm.at[posv1], sem)
        c2 = pltpu.async_copy(xbuf, xs_hbm.at[posv2], sem)
        c1.wait()
        c2.wait()


def _sc_scatter(x_flat, pos1, pos2):
    return pl.kernel(
        _scatter_body,
        out_type=jax.ShapeDtypeStruct((_GMAX, _D), jnp.float32),
        mesh=_sc_mesh(),
        scratch_types=[
            pltpu.VMEM((_CC,), jnp.int32),
            pltpu.VMEM((_CC,), jnp.int32),
            pltpu.VMEM((_CC, _D), jnp.float32),
            pltpu.SemaphoreType.DMA,
        ],
        interpret=_INTERPRET,
    )(x_flat, pos1, pos2)


def _combine_body(ys_hbm, pos1_hbm, pos2_hbm, p1_hbm, p2_hbm, out_hbm,
                  posv1, posv2, pv1, pv2, buf1, buf2, sem):
    wid = lax.axis_index("s") * 2 + lax.axis_index("c")
    base = wid * _TW
    for ch in range(_NCH):
        b = base + ch * _CC
        pltpu.sync_copy(pos1_hbm.at[pl.ds(b, _CC)], posv1)
        pltpu.sync_copy(pos2_hbm.at[pl.ds(b, _CC)], posv2)
        pltpu.sync_copy(p1_hbm.at[pl.ds(b, _CC)], pv1.at[pl.ds(0, _CC)])
        pltpu.sync_copy(p2_hbm.at[pl.ds(b, _CC)], pv2.at[pl.ds(0, _CC)])
        g1 = pltpu.async_copy(ys_hbm.at[posv1], buf1, sem)
        g2 = pltpu.async_copy(ys_hbm.at[posv2], buf2, sem)
        g1.wait()
        g2.wait()

        def _row(r, _):
            s1 = pv1[pl.ds(r, 16)][0]
            s2 = pv2[pl.ds(r, 16)][0]
            for c in range(_D // 16):
                sl = pl.ds(c * 16, 16)
                buf1[r, sl] = buf1[r, sl] * s1 + buf2[r, sl] * s2
            return 0

        lax.fori_loop(0, _CC, _row, 0)
        pltpu.sync_copy(buf1, out_hbm.at[pl.ds(b, _CC)])


def _sc_combine(ys, pos1, pos2, p1, p2):
    return pl.kernel(
        _combine_body,
        out_type=jax.ShapeDtypeStruct((_M, _D), jnp.float32),
        mesh=_sc_mesh(),
        scratch_types=[
            pltpu.VMEM((_CC,), jnp.int32),
            pltpu.VMEM((_CC,), jnp.int32),
            pltpu.VMEM((_CC + 16,), jnp.float32),
            pltpu.VMEM((_CC + 16,), jnp.float32),
            pltpu.VMEM((_CC, _D), jnp.float32),
            pltpu.VMEM((_CC, _D), jnp.float32),
            pltpu.SemaphoreType.DMA,
        ],
        interpret=_INTERPRET,
    )(ys, pos1, pos2, p1, p2)


def kernel(x, Wg, bg, W1, b1, W2, b2):
    x_flat = x.reshape(_M, _D)
    i1, i2, p1, p2, load, lbl = _routing(x_flat, Wg, bg)
    pos1, pos2, te = _dispatch(i1, i2)
    pos1, pos2 = pos1.reshape(_M), pos2.reshape(_M)
    xs = _sc_scatter(x_flat, pos1, pos2)
    ys = _gmm(xs, W1, b1, W2, b2,
              te.reshape(_GT))
    combined = _sc_combine(ys, pos1, pos2, p1.reshape(_M), p2.reshape(_M))

    return (combined.reshape(_B, _S, _D), lbl.reshape(()), load.reshape(_E))
